# baseline (speedup 1.0000x reference)
import functools

import jax
import jax.numpy as jnp
from jax import lax
from jax.experimental import pallas as pl
from jax.experimental.pallas import tpu as pltpu
from jax.experimental.pallas import tpu_sc as plsc

N = 10000
L = 10
NPL = N // L
E = 320000
D = 128

G = 8
NQ = 4
DQ = D // NQ
CH = 1024
NCH = 36
EP = NCH * CH
NEG = float(jnp.finfo(jnp.float32).min)

_LANES = 16
_SUBLANES = DQ // _LANES


def _worker_body(featq_hbm, src_hbm, dst_hbm, out_hbm, acc0, acc1, rows, srcb,
                 dsts, dstm, sem):
  c = lax.axis_index("c")
  s = lax.axis_index("s")
  wid = c * 16 + s
  lay = wid % G
  q = wid // G

  cbase = -(lay + 1) * (NPL * DQ)
  col0 = lax.iota(jnp.int32, _LANES)
  neg16 = jnp.full((_LANES,), NEG, jnp.float32)

  def init_row(r, _):
    for u in range(8):
      acc0[pl.ds((r * 8 + u) * _LANES, _LANES)] = neg16
      acc1[pl.ds((r * 8 + u) * _LANES, _LANES)] = neg16
    return _
  lax.fori_loop(0, NPL * _SUBLANES // 8, init_row, 0)

  def chunk_body(b, _):
    pltpu.sync_copy(src_hbm.at[lay, pl.ds(b * (CH // 128), CH // 128)], srcb)
    pltpu.sync_copy(dst_hbm.at[lay, pl.ds(b * CH, CH)], dstm.at[s])
    pltpu.sync_copy(dstm.at[s], dsts)

    def idx_body(i, _):
      for j in range(8):
        v = srcb[i, pl.ds(j * _LANES, _LANES)]
        srcb[i, pl.ds(j * _LANES, _LANES)] = v * NQ + q
      return _
    lax.fori_loop(0, CH // 128, idx_body, 0)

    cps = [
        pltpu.async_copy(featq_hbm.at[srcb.at[i]],
                         rows.at[pl.ds(i * 128, 128)], sem)
        for i in range(CH // 128)
    ]
    for cp in cps:
      cp.wait()

    def group_body(g, _):
      base = g * _LANES
      for j in range(_LANES):
        acc = acc0 if j % 2 == 0 else acc1
        off = dsts[base + j] * DQ + cbase
        r0 = rows[base + j, pl.ds(0, _LANES)]
        r1 = rows[base + j, pl.ds(_LANES, _LANES)]
        a0 = acc[pl.ds(off, _LANES)]
        a1 = acc[pl.ds(off + _LANES, _LANES)]
        acc[pl.ds(off, _LANES)] = jnp.maximum(a0, r0)
        acc[pl.ds(off + _LANES, _LANES)] = jnp.maximum(a1, r1)
      return _
    lax.fori_loop(0, CH // _LANES, group_body, 0)
    return _

  lax.fori_loop(0, NCH, chunk_body, 0)

  def fix_row(r, _):
    for h in range(_SUBLANES):
      off = (r * _SUBLANES + h) * _LANES
      v = jnp.maximum(acc0[pl.ds(off, _LANES)], acc1[pl.ds(off, _LANES)])
      rows[r, pl.ds(h * _LANES, _LANES)] = jnp.where(v == neg16, 0.0, v)
    return _
  lax.fori_loop(0, NPL, fix_row, 0)
  pltpu.sync_copy(rows.at[pl.ds(0, NPL)],
                  out_hbm.at[pl.ds((lay + 1) * NPL, NPL), q, :])

  @pl.when(lay == 0)
  def _():
    def zero_row(r, _):
      for h in range(_SUBLANES):
        rows[r, pl.ds(h * _LANES, _LANES)] = jnp.zeros((_LANES,), jnp.float32)
      return _
    lax.fori_loop(0, NPL, zero_row, 0)
    pltpu.sync_copy(rows.at[pl.ds(0, NPL)], out_hbm.at[pl.ds(0, NPL), q, :])

  @pl.when(lay == 1)
  def _():
    kmax = jnp.full((_LANES,), ((N - 1) * NQ + 3), jnp.int32)
    def gen_body(i, _):
      for j in range(8):
        bval = 9 * NPL * NQ + q + (i * 128 + j * _LANES) * NQ
        v = jnp.minimum(bval + col0 * NQ, kmax - (3 - q))
        srcb[i, pl.ds(j * _LANES, _LANES)] = v
      return _
    lax.fori_loop(0, 8, gen_body, 0)
    cps = [
        pltpu.async_copy(featq_hbm.at[srcb.at[i]],
                         rows.at[pl.ds(i * 128, 128)], sem)
        for i in range(8)
    ]
    for cp in cps:
      cp.wait()
    pltpu.sync_copy(rows.at[pl.ds(0, NPL)],
                    out_hbm.at[pl.ds((L - 1) * NPL, NPL), q, :])


@jax.jit
def kernel(feat, edge_index):
  ei = jnp.concatenate(
      [edge_index, edge_index[:, 5:9], edge_index[:, :EP * 9 - E - 4]], axis=1)
  eiT = ei.reshape(2, EP, 9).transpose(0, 2, 1)
  srcT = eiT[0, :G].reshape(G, EP // 128, 128)
  dstT = eiT[1, :G]
  featq = feat.reshape(NQ * N, DQ)

  mesh = plsc.VectorSubcoreMesh(core_axis_name="c", subcore_axis_name="s")
  run = functools.partial(
      pl.kernel,
      out_type=jax.ShapeDtypeStruct((N, NQ, DQ), jnp.float32),
      mesh=mesh,
      compiler_params=pltpu.CompilerParams(
          needs_layout_passes=False, use_tc_tiling_on_sc=False),
      scratch_types=[
          pltpu.VMEM((NPL * DQ,), jnp.float32),
          pltpu.VMEM((NPL * DQ,), jnp.float32),
          pltpu.VMEM((CH, DQ), jnp.float32),
          pltpu.VMEM((CH // 128, 128), jnp.int32),
          pltpu.SMEM((CH,), jnp.int32),
          pltpu.VMEM_SHARED((16, CH), jnp.int32),
          pltpu.SemaphoreType.DMA,
      ],
  )(_worker_body)
  outq = run(featq, srcT, dstT)
  return outq.reshape(N, D)


# speedup vs baseline: 1.2641x; 1.2641x over previous
import functools

import jax
import jax.numpy as jnp
from jax import lax
from jax.experimental import pallas as pl
from jax.experimental.pallas import tpu as pltpu
from jax.experimental.pallas import tpu_sc as plsc

N = 10000
L = 10
NPL = N // L
E = 320000
D = 128

G = 8
NQ = 4
DQ = D // NQ
CH = 1024
NCH = 36
EP = NCH * CH
NEG = float(jnp.finfo(jnp.float32).min)

_LANES = 16
_SUBLANES = DQ // _LANES


def _worker_body(featq_hbm, src_hbm, dst_hbm, out_hbm, acc0, acc1, rows, srcb,
                 dsts, dstm, sem):
  c = lax.axis_index("c")
  s = lax.axis_index("s")
  wid = c * 16 + s
  lay = wid % G
  q = wid // G

  cbase = -(lay + 1) * (NPL * DQ)
  col0 = lax.iota(jnp.int32, _LANES)
  neg16 = jnp.full((_LANES,), NEG, jnp.float32)

  def init_row(r, _):
    for u in range(8):
      acc0[pl.ds((r * 8 + u) * _LANES, _LANES)] = neg16
      acc1[pl.ds((r * 8 + u) * _LANES, _LANES)] = neg16
    return _
  lax.fori_loop(0, NPL * _SUBLANES // 8, init_row, 0)

  def chunk_body(b, _):
    pltpu.sync_copy(src_hbm.at[lay, pl.ds(b * (CH // 128), CH // 128)], srcb)
    pltpu.sync_copy(dst_hbm.at[lay, pl.ds(b * CH, CH)], dstm.at[s])
    pltpu.sync_copy(dstm.at[s], dsts)

    def idx_body(i, _):
      for j in range(8):
        v = srcb[i, pl.ds(j * _LANES, _LANES)]
        srcb[i, pl.ds(j * _LANES, _LANES)] = v * NQ + q
      return _
    lax.fori_loop(0, CH // 128, idx_body, 0)

    cps = [
        pltpu.async_copy(featq_hbm.at[srcb.at[i]],
                         rows.at[pl.ds(i * 128, 128)], sem)
        for i in range(CH // 128)
    ]
    for cp in cps:
      cp.wait()

    def eload(base, j):
      acc = acc0 if j % 2 == 0 else acc1
      off = dsts[base + j] * DQ + cbase
      a0 = acc[pl.ds(off, _LANES)]
      a1 = acc[pl.ds(off + _LANES, _LANES)]
      r0 = rows[base + j, pl.ds(0, _LANES)]
      r1 = rows[base + j, pl.ds(_LANES, _LANES)]
      return (acc, off, a0, a1, r0, r1)

    def efin(t):
      acc, off, a0, a1, r0, r1 = t
      acc[pl.ds(off, _LANES)] = jnp.maximum(a0, r0)
      acc[pl.ds(off + _LANES, _LANES)] = jnp.maximum(a1, r1)

    def group_body(g, _):
      base = g * _LANES
      t = eload(base, 0)
      for j in range(1, _LANES):
        t2 = eload(base, j)
        efin(t)
        t = t2
      efin(t)
      return _
    lax.fori_loop(0, CH // _LANES, group_body, 0)
    return _

  lax.fori_loop(0, NCH, chunk_body, 0)

  def fix_row(r, _):
    for h in range(_SUBLANES):
      off = (r * _SUBLANES + h) * _LANES
      v = jnp.maximum(acc0[pl.ds(off, _LANES)], acc1[pl.ds(off, _LANES)])
      rows[r, pl.ds(h * _LANES, _LANES)] = jnp.where(v == neg16, 0.0, v)
    return _
  lax.fori_loop(0, NPL, fix_row, 0)
  pltpu.sync_copy(rows.at[pl.ds(0, NPL)],
                  out_hbm.at[pl.ds((lay + 1) * NPL, NPL), q, :])

  @pl.when(lay == 0)
  def _():
    def zero_row(r, _):
      for h in range(_SUBLANES):
        rows[r, pl.ds(h * _LANES, _LANES)] = jnp.zeros((_LANES,), jnp.float32)
      return _
    lax.fori_loop(0, NPL, zero_row, 0)
    pltpu.sync_copy(rows.at[pl.ds(0, NPL)], out_hbm.at[pl.ds(0, NPL), q, :])

  @pl.when(lay == 1)
  def _():
    kmax = jnp.full((_LANES,), ((N - 1) * NQ + 3), jnp.int32)
    def gen_body(i, _):
      for j in range(8):
        bval = 9 * NPL * NQ + q + (i * 128 + j * _LANES) * NQ
        v = jnp.minimum(bval + col0 * NQ, kmax - (3 - q))
        srcb[i, pl.ds(j * _LANES, _LANES)] = v
      return _
    lax.fori_loop(0, 8, gen_body, 0)
    cps = [
        pltpu.async_copy(featq_hbm.at[srcb.at[i]],
                         rows.at[pl.ds(i * 128, 128)], sem)
        for i in range(8)
    ]
    for cp in cps:
      cp.wait()
    pltpu.sync_copy(rows.at[pl.ds(0, NPL)],
                    out_hbm.at[pl.ds((L - 1) * NPL, NPL), q, :])


@jax.jit
def kernel(feat, edge_index):
  ei = jnp.concatenate(
      [edge_index, edge_index[:, 5:9], edge_index[:, :EP * 9 - E - 4]], axis=1)
  eiT = ei.reshape(2, EP, 9).transpose(0, 2, 1)
  srcT = eiT[0, :G].reshape(G, EP // 128, 128)
  dstT = eiT[1, :G]
  featq = feat.reshape(NQ * N, DQ)

  mesh = plsc.VectorSubcoreMesh(core_axis_name="c", subcore_axis_name="s")
  run = functools.partial(
      pl.kernel,
      out_type=jax.ShapeDtypeStruct((N, NQ, DQ), jnp.float32),
      mesh=mesh,
      compiler_params=pltpu.CompilerParams(
          needs_layout_passes=False, use_tc_tiling_on_sc=False),
      scratch_types=[
          pltpu.VMEM((NPL * DQ,), jnp.float32),
          pltpu.VMEM((NPL * DQ,), jnp.float32),
          pltpu.VMEM((CH, DQ), jnp.float32),
          pltpu.VMEM((CH // 128, 128), jnp.int32),
          pltpu.SMEM((CH,), jnp.int32),
          pltpu.VMEM_SHARED((16, CH), jnp.int32),
          pltpu.SemaphoreType.DMA,
      ],
  )(_worker_body)
  outq = run(featq, srcT, dstT)
  return outq.reshape(N, D)


# speedup vs baseline: 1.6716x; 1.3224x over previous
import functools

import jax
import jax.numpy as jnp
from jax import lax
from jax.experimental import pallas as pl
from jax.experimental.pallas import tpu as pltpu
from jax.experimental.pallas import tpu_sc as plsc

N = 10000
L = 10
NPL = N // L
E = 320000
D = 128

G = 8
NQ = 4
DQ = D // NQ
CH = 768
NCH = 48
EP = NCH * CH
NEG = float(jnp.finfo(jnp.float32).min)

_LANES = 16
_SUBLANES = DQ // _LANES
_NG = CH // 128


def _worker_body(featq_hbm, src_hbm, dst_hbm, out_hbm, acc0, acc1, rows, srcb,
                 dsts, dstm, semS, semD, semG):
  c = lax.axis_index("c")
  s = lax.axis_index("s")
  wid = c * 16 + s
  lay = wid % G
  q = wid // G

  rbase = -(lay + 1) * NPL
  col0 = lax.iota(jnp.int32, _LANES)
  neg16 = jnp.full((_LANES,), NEG, jnp.float32)

  def init_row(r, _):
    for h in range(_SUBLANES):
      acc0[r, pl.ds(h * _LANES, _LANES)] = neg16
      acc1[r, pl.ds(h * _LANES, _LANES)] = neg16
    return _
  lax.fori_loop(0, NPL, init_row, 0)

  def issue_stage(b):
    sl = b % 3
    pltpu.async_copy(src_hbm.at[lay, pl.ds(b * _NG, _NG)], srcb.at[sl],
                     semS.at[sl])
    pltpu.async_copy(dst_hbm.at[lay, pl.ds(b * CH, CH)], dstm.at[s, sl],
                     semD.at[sl])

  def fire_gathers(b):
    sl = b % 3
    p = b % 2
    pltpu.make_async_copy(src_hbm.at[lay, pl.ds(b * _NG, _NG)], srcb.at[sl],
                          semS.at[sl]).wait()
    def idx_body(i, _):
      for j in range(8):
        v = srcb[sl, i, pl.ds(j * _LANES, _LANES)]
        srcb[sl, i, pl.ds(j * _LANES, _LANES)] = v * NQ + q
      return _
    lax.fori_loop(0, _NG, idx_body, 0)
    for i in range(_NG):
      pltpu.async_copy(featq_hbm.at[srcb.at[sl, i]],
                       rows.at[p, pl.ds(i * 128, 128)], semG.at[p])

  def chunk_body(b, carry):
    sl = b % 3
    p = b % 2

    @pl.when(b + 2 < NCH)
    def _prefetch():
      issue_stage(b + 2)

    @pl.when(b + 1 < NCH)
    def _fire():
      fire_gathers(b + 1)

    for i in range(_NG):
      pltpu.make_async_copy(featq_hbm.at[srcb.at[sl, i]],
                            rows.at[p, pl.ds(i * 128, 128)],
                            semG.at[p]).wait()
    pltpu.make_async_copy(dst_hbm.at[lay, pl.ds(b * CH, CH)], dstm.at[s, sl],
                          semD.at[sl]).wait()
    pltpu.sync_copy(dstm.at[s, sl], dsts)

    def eload(base, j):
      acc = acc0 if j % 2 == 0 else acc1
      row = dsts[base + j] + rbase
      a0 = acc[row, pl.ds(0, _LANES)]
      a1 = acc[row, pl.ds(_LANES, _LANES)]
      r0 = rows[p, base + j, pl.ds(0, _LANES)]
      r1 = rows[p, base + j, pl.ds(_LANES, _LANES)]
      return (acc, row, a0, a1, r0, r1)

    def efin(t):
      acc, row, a0, a1, r0, r1 = t
      acc[row, pl.ds(0, _LANES)] = jnp.maximum(a0, r0)
      acc[row, pl.ds(_LANES, _LANES)] = jnp.maximum(a1, r1)

    def group_body(g, _):
      base = g * _LANES
      t = eload(base, 0)
      for j in range(1, _LANES):
        t2 = eload(base, j)
        efin(t)
        t = t2
      efin(t)
      return _
    lax.fori_loop(0, CH // _LANES, group_body, 0)
    return carry

  issue_stage(0)
  issue_stage(1)
  fire_gathers(0)
  lax.fori_loop(0, NCH, chunk_body, 0)

  def fix_row(r, _):
    for h in range(_SUBLANES):
      v = jnp.maximum(acc0[r, pl.ds(h * _LANES, _LANES)],
                      acc1[r, pl.ds(h * _LANES, _LANES)])
      acc0[r, pl.ds(h * _LANES, _LANES)] = jnp.where(v == neg16, 0.0, v)
    return _
  lax.fori_loop(0, NPL, fix_row, 0)
  pltpu.sync_copy(acc0, out_hbm.at[pl.ds((lay + 1) * NPL, NPL), q, :])

  @pl.when(lay == 0)
  def _():
    def zero_row(r, _):
      for h in range(_SUBLANES):
        acc1[r, pl.ds(h * _LANES, _LANES)] = jnp.zeros(
            (_LANES,), jnp.float32)
      return _
    lax.fori_loop(0, NPL, zero_row, 0)
    pltpu.sync_copy(acc1, out_hbm.at[pl.ds(0, NPL), q, :])

  @pl.when(lay == 1)
  def _():
    kcap = jnp.full((_LANES,), (N - 1) * NQ + q, jnp.int32)
    for t in range(8):
      for j in range(8):
        bval = 9 * NPL * NQ + q + (t * 128 + j * _LANES) * NQ
        v = jnp.minimum(bval + col0 * NQ, kcap)
        srcb[t // _NG, t % _NG, pl.ds(j * _LANES, _LANES)] = v
    cps = []
    for t in range(7):
      cps.append(
          pltpu.async_copy(featq_hbm.at[srcb.at[t // _NG, t % _NG]],
                           acc1.at[pl.ds(t * 128, 128)], semG.at[0]))
    cps.append(
        pltpu.async_copy(
            featq_hbm.at[srcb.at[7 // _NG, 7 % _NG, pl.ds(0, NPL - 896)]],
            acc1.at[pl.ds(896, NPL - 896)], semG.at[0]))
    for cp in cps:
      cp.wait()
    pltpu.sync_copy(acc1, out_hbm.at[pl.ds((L - 1) * NPL, NPL), q, :])


@jax.jit
def kernel(feat, edge_index):
  ei = jnp.concatenate(
      [edge_index, edge_index[:, 5:9], edge_index[:, :EP * 9 - E - 4]], axis=1)
  eiT = ei.reshape(2, EP, 9).transpose(0, 2, 1)
  srcT = eiT[0, :G].reshape(G, EP // 128, 128)
  dstT = eiT[1, :G]
  featq = feat.reshape(NQ * N, DQ)

  mesh = plsc.VectorSubcoreMesh(core_axis_name="c", subcore_axis_name="s")
  run = functools.partial(
      pl.kernel,
      out_type=jax.ShapeDtypeStruct((N, NQ, DQ), jnp.float32),
      mesh=mesh,
      compiler_params=pltpu.CompilerParams(
          needs_layout_passes=False, use_tc_tiling_on_sc=False),
      scratch_types=[
          pltpu.VMEM((NPL, DQ), jnp.float32),
          pltpu.VMEM((NPL, DQ), jnp.float32),
          pltpu.VMEM((2, CH, DQ), jnp.float32),
          pltpu.VMEM((3, _NG, 128), jnp.int32),
          pltpu.SMEM((CH,), jnp.int32),
          pltpu.VMEM_SHARED((16, 3, CH), jnp.int32),
          pltpu.SemaphoreType.DMA((3,)),
          pltpu.SemaphoreType.DMA((3,)),
          pltpu.SemaphoreType.DMA((2,)),
      ],
  )(_worker_body)
  outq = run(featq, srcT, dstT)
  return outq.reshape(N, D)
